# baseline (speedup 1.0000x reference)
import functools

import jax
import jax.numpy as jnp
from jax import lax
from jax.experimental import pallas as pl
from jax.experimental.pallas import tpu as pltpu
from jax.experimental.pallas import tpu_sc as plsc

N_ROWS = 320000
D_IN = 128
D_H = 256
N_SEG = 10000

BLK = 16000


def _mlp_body(x_ref, s_ref, w0_ref, w1_ref, w2_ref, w3_ref, o_ref):
    bf = jnp.bfloat16
    h = jnp.dot(x_ref[...].astype(bf), w0_ref[...].astype(bf),
                preferred_element_type=jnp.float32)
    h = jnp.dot(jnp.maximum(h, 0.0).astype(bf), w1_ref[...].astype(bf),
                preferred_element_type=jnp.float32)
    h = jnp.dot(jnp.maximum(h, 0.0).astype(bf), w2_ref[...].astype(bf),
                preferred_element_type=jnp.float32)
    logits = jnp.dot(jnp.maximum(h, 0.0).astype(bf), w3_ref[...].astype(bf),
                     preferred_element_type=jnp.float32)
    o_ref[...] = (logits.reshape(BLK // 128, 128) * s_ref[0])[None]


def _mlp(X, scores, W0, W1, W2, W3, i0, nblk):
    return pl.pallas_call(
        _mlp_body,
        grid=(nblk,),
        in_specs=[
            pl.BlockSpec((BLK, D_IN), lambda i: (i + i0, 0)),
            pl.BlockSpec((1, BLK // 128, 128), lambda i: (i + i0, 0, 0)),
            pl.BlockSpec((D_IN, D_H), lambda i: (0, 0)),
            pl.BlockSpec((D_H, D_H), lambda i: (0, 0)),
            pl.BlockSpec((D_H, D_H), lambda i: (0, 0)),
            pl.BlockSpec((D_H, 1), lambda i: (0, 0)),
        ],
        out_specs=pl.BlockSpec((1, BLK // 128, 128), lambda i: (i, 0, 0)),
        out_shape=jax.ShapeDtypeStruct((nblk, BLK // 128, 128),
                                       jnp.float32),
        compiler_params=pltpu.CompilerParams(
            dimension_semantics=("parallel",)),
    )(X, scores, W0, W1, W2, W3)


NW = 32
NS = 16
ACC = 10240

@functools.cache
def _build_segsum(rpw):
    mesh = plsc.VectorSubcoreMesh(core_axis_name="c", subcore_axis_name="s")
    body = functools.partial(_segsum_body, rpw)
    return functools.partial(
        pl.kernel,
        mesh=mesh,
        out_type=jax.ShapeDtypeStruct((2 * ACC,), jnp.float32),
        scratch_types=[
            pltpu.VMEM((rpw, 128), jnp.float32),
            pltpu.VMEM((rpw, 128), jnp.int32),
            pltpu.VMEM((ACC // NS, ), jnp.float32),
            pltpu.VMEM_SHARED((ACC,), jnp.float32),
            pltpu.SemaphoreType.DMA,
            pltpu.SemaphoreType.DMA,
        ],
    )(body)


def _segsum_body(rpw, w_hbm, idx_hbm, out_hbm, w_v, idx_v, zrow_v, acc_sh,
                 sem, lsem):
    c = lax.axis_index("c")
    s = lax.axis_index("s")
    gid = s * 2 + c

    row0 = pl.multiple_of(gid * rpw, rpw)
    w_load = pltpu.async_copy(w_hbm.at[pl.ds(row0, rpw)], w_v, lsem)
    i_load = pltpu.async_copy(idx_hbm.at[pl.ds(row0, rpw)], idx_v, lsem)

    stripe = ACC // NS

    def zfill(j, carry):
        zrow_v[pl.ds(pl.multiple_of(j * 16, 16), 16)] = jnp.zeros(
            (16,), jnp.float32)
        return carry

    lax.fori_loop(0, stripe // 16, zfill, 0)
    pltpu.sync_copy(zrow_v, acc_sh.at[pl.ds(pl.multiple_of(s * stripe, 8),
                                            stripe)])

    w_load.wait()
    i_load.wait()
    plsc.subcore_barrier()

    KB = 8

    def body(g, carry):
        copies = [
            pltpu.async_copy(w_v.at[g * KB + b],
                             acc_sh.at[idx_v.at[g * KB + b]], sem, add=True)
            for b in range(KB)
        ]
        for cp in copies:
            cp.wait()
        return carry

    lax.fori_loop(0, rpw // KB, body, 0)

    plsc.subcore_barrier()

    @pl.when(s == 0)
    def _writeback():
        off = pl.multiple_of(c * ACC, ACC)
        pltpu.sync_copy(acc_sh, out_hbm.at[pl.ds(off, ACC)])


def kernel(X, ppr_scores, ppr_idx, W0, W1, W2, W3):
    nblk = N_ROWS // BLK
    hblk = nblk // 2
    hrows = N_ROWS // 2 // 128
    rpw = 40
    hpad = NW * rpw
    scores3 = ppr_scores.reshape(nblk, BLK // 128, 128)
    idx32 = ppr_idx.astype(jnp.int32)

    def half(i0):
        w = _mlp(X, scores3, W0, W1, W2, W3, i0 * hblk, hblk)
        w_pad = jnp.pad(w.reshape(hrows, 128), ((0, hpad - hrows), (0, 0)))
        idx = lax.dynamic_slice_in_dim(idx32, i0 * (N_ROWS // 2), N_ROWS // 2)
        idx_pad = jnp.pad(idx, (0, (hpad - hrows) * 128),
                          constant_values=N_SEG).reshape(hpad, 128)
        return _build_segsum(rpw)(w_pad, idx_pad)

    pa = half(0)
    pb = half(1)
    return (pa[:ACC] + pa[ACC:] + pb[:ACC] + pb[ACC:])[:N_SEG, None]
